# baseline (device time: 66603 ns/iter reference)
import jax
import jax.numpy as jnp
from jax import lax
from jax.experimental import pallas as pl
from jax.experimental.pallas import tpu as pltpu

CH = 4
C = 4
B = 10


def kernel(x, w_mat):
    k_glob, kc = x.shape
    n_dev = k_glob // kc
    m_per = kc
    _, n = w_mat.shape
    ks = CH * kc
    nc = n // C
    P = n_dev // CH
    n_slab = P * C

    def body(x_ref, w_ref, out_ref, xg_ref, amax_ref, wbuf, wsem,
             xs_sem, xr_sem, as_sem, ar_sem):
        me = lax.axis_index("i")

        def x_rdma(off):
            d = lax.rem(me + n_dev - off, n_dev)
            return pltpu.make_async_remote_copy(
                src_ref=x_ref.at[pl.ds(d * m_per, m_per), :],
                dst_ref=xg_ref.at[:, pl.ds(off * kc, kc)],
                send_sem=xs_sem.at[off],
                recv_sem=xr_sem.at[off],
                device_id=(d,),
                device_id_type=pl.DeviceIdType.MESH,
            )

        def x_recv(t):
            return pltpu.make_async_remote_copy(
                src_ref=x_ref.at[pl.ds(0, m_per), :],
                dst_ref=xg_ref.at[:, pl.ds(t * kc, kc)],
                send_sem=xs_sem.at[t],
                recv_sem=xr_sem.at[t],
                device_id=(me,),
                device_id_type=pl.DeviceIdType.MESH,
            )

        def w_dmas(i, slot):
            p, c = divmod(i, C)
            copies = []
            for q in range(CH):
                r = lax.rem(me + CH * p + q, n_dev) * kc
                copies.append(pltpu.make_async_copy(
                    w_ref.at[pl.ds(r, kc), pl.ds(c * nc, nc)],
                    wbuf.at[slot, pl.ds(q * kc, kc), :],
                    wsem.at[slot, q],
                ))
            return copies

        xg_ref[:, pl.ds(0, kc)] = x_ref[pl.ds(me * m_per, m_per), :]
        for i in range(B):
            for cp in w_dmas(i, i):
                cp.start()

        for off in range(1, n_dev):
            x_rdma(off).start()

        barrier_sem = pltpu.get_barrier_semaphore()
        for d in (lax.rem(me + 1, n_dev), lax.rem(me + n_dev - 1, n_dev)):
            pl.semaphore_signal(barrier_sem, inc=1, device_id=(d,),
                                device_id_type=pl.DeviceIdType.MESH)
        pl.semaphore_wait(barrier_sem, 2)

        chunk_amax = []
        for i in range(n_slab):
            p, c = divmod(i, C)
            if c == 0:
                for q in range(CH):
                    t = CH * p + q
                    if t > 0:
                        x_recv(t).wait_recv()
            slot = i % B
            for cp in w_dmas(i, slot):
                cp.wait()
            a_op = xg_ref[:, p * ks:(p + 1) * ks]
            prod = jnp.dot(a_op, wbuf[slot],
                           preferred_element_type=jnp.float32)
            if p == 0:
                out_ref[:, c * nc:(c + 1) * nc] = prod
            else:
                acc = out_ref[:, c * nc:(c + 1) * nc] + prod
                out_ref[:, c * nc:(c + 1) * nc] = acc
                if p == P - 1:
                    chunk_amax.append(jnp.max(jnp.abs(acc)))
            if i + B < n_slab:
                for cp in w_dmas(i + B, slot):
                    cp.start()

        for off in range(1, n_dev):
            x_rdma(off).wait_send()

        local_amax = jnp.maximum(jnp.maximum(chunk_amax[0], chunk_amax[1]),
                                 jnp.maximum(chunk_amax[2], chunk_amax[3]))
        amax_ref[pl.ds(me, 1)] = jnp.full((1, 8, 128), local_amax,
                                          jnp.float32)

        def a_rdma(d, src_slot):
            return pltpu.make_async_remote_copy(
                src_ref=amax_ref.at[me],
                dst_ref=amax_ref.at[src_slot],
                send_sem=as_sem.at[d],
                recv_sem=ar_sem.at[src_slot],
                device_id=(d,),
                device_id_type=pl.DeviceIdType.MESH,
            )

        for off in range(1, n_dev):
            d = lax.rem(me + off, n_dev)
            a_rdma(d, me).start()
        for off in range(1, n_dev):
            s = lax.rem(me + off, n_dev)
            a_rdma(me, s).wait_recv()
        for off in range(1, n_dev):
            d = lax.rem(me + off, n_dev)
            a_rdma(d, me).wait_send()

        g_amax = jnp.max(amax_ref[:, :, :])
        scale = g_amax / 448.0
        inv = 448.0 / g_amax
        q = (out_ref[:, :] * inv).astype(jnp.float8_e4m3fn)
        out_ref[:, :] = q.astype(jnp.float32) * scale

    return pl.pallas_call(
        body,
        out_shape=jax.ShapeDtypeStruct((m_per, n), jnp.float32),
        in_specs=[
            pl.BlockSpec(memory_space=pltpu.VMEM),
            pl.BlockSpec(memory_space=pl.ANY),
        ],
        out_specs=pl.BlockSpec(memory_space=pltpu.VMEM),
        scratch_shapes=[
            pltpu.VMEM((m_per, k_glob), jnp.float32),
            pltpu.VMEM((n_dev, 8, 128), jnp.float32),
            pltpu.VMEM((B, ks, nc), jnp.float32),
            pltpu.SemaphoreType.DMA((B, CH)),
            pltpu.SemaphoreType.DMA((n_dev,)),
            pltpu.SemaphoreType.DMA((n_dev,)),
            pltpu.SemaphoreType.DMA((n_dev,)),
            pltpu.SemaphoreType.DMA((n_dev,)),
        ],
        compiler_params=pltpu.CompilerParams(
            vmem_limit_bytes=100 * 1024 * 1024,
            collective_id=0,
        ),
    )(x, w_mat)


# device time: 64561 ns/iter; 1.0316x vs baseline; 1.0316x over previous
import jax
import jax.numpy as jnp
from jax import lax
from jax.experimental import pallas as pl
from jax.experimental.pallas import tpu as pltpu

CH = 4
C = 4
B = 10


def kernel(x, w_mat):
    k_glob, kc = x.shape
    n_dev = k_glob // kc
    m_per = kc
    _, n = w_mat.shape
    ks = CH * kc
    nc = n // C
    P = n_dev // CH
    n_slab = P * C

    def body(x_ref, w_ref, out_ref, xg_ref, amax_ref, wbuf, wsem,
             xs_sem, xr_sem, as_sem, ar_sem):
        me = lax.axis_index("i")

        def x_rdma(off):
            d = lax.rem(me + n_dev - off, n_dev)
            return pltpu.make_async_remote_copy(
                src_ref=x_ref.at[pl.ds(d * m_per, m_per), :],
                dst_ref=xg_ref.at[:, pl.ds(off * kc, kc)],
                send_sem=xs_sem.at[off],
                recv_sem=xr_sem.at[off],
                device_id=(d,),
                device_id_type=pl.DeviceIdType.MESH,
            )

        def x_recv(t):
            return pltpu.make_async_remote_copy(
                src_ref=x_ref.at[pl.ds(0, m_per), :],
                dst_ref=xg_ref.at[:, pl.ds(t * kc, kc)],
                send_sem=xs_sem.at[t],
                recv_sem=xr_sem.at[t],
                device_id=(me,),
                device_id_type=pl.DeviceIdType.MESH,
            )

        def w_dmas(i, slot):
            p, c = divmod(i, C)
            copies = []
            for q in range(CH):
                r = lax.rem(me + CH * p + q, n_dev) * kc
                copies.append(pltpu.make_async_copy(
                    w_ref.at[pl.ds(r, kc), pl.ds(c * nc, nc)],
                    wbuf.at[slot, pl.ds(q * kc, kc), :],
                    wsem.at[slot, q],
                ))
            return copies

        xg_ref[:, pl.ds(0, kc)] = x_ref[pl.ds(me * m_per, m_per), :]
        for i in range(B):
            for cp in w_dmas(i, i):
                cp.start()

        barrier_sem = pltpu.get_barrier_semaphore()
        for off in range(1, n_dev):
            d = lax.rem(me + off, n_dev)
            pl.semaphore_signal(barrier_sem, inc=1, device_id=(d,),
                                device_id_type=pl.DeviceIdType.MESH)
        pl.semaphore_wait(barrier_sem, n_dev - 1)

        for off in range(1, n_dev):
            x_rdma(off).start()

        chunk_amax = []
        for i in range(n_slab):
            p, c = divmod(i, C)
            if c == 0:
                for q in range(CH):
                    t = CH * p + q
                    if t > 0:
                        x_recv(t).wait_recv()
            slot = i % B
            for cp in w_dmas(i, slot):
                cp.wait()
            a_op = xg_ref[:, p * ks:(p + 1) * ks]
            prod = jnp.dot(a_op, wbuf[slot],
                           preferred_element_type=jnp.float32)
            if p == 0:
                out_ref[:, c * nc:(c + 1) * nc] = prod
            else:
                acc = out_ref[:, c * nc:(c + 1) * nc] + prod
                out_ref[:, c * nc:(c + 1) * nc] = acc
                if p == P - 1:
                    chunk_amax.append(jnp.max(jnp.abs(acc)))
            if i + B < n_slab:
                for cp in w_dmas(i + B, slot):
                    cp.start()

        for off in range(1, n_dev):
            x_rdma(off).wait_send()

        local_amax = jnp.maximum(jnp.maximum(chunk_amax[0], chunk_amax[1]),
                                 jnp.maximum(chunk_amax[2], chunk_amax[3]))
        amax_ref[pl.ds(me, 1)] = jnp.full((1, 8, 128), local_amax,
                                          jnp.float32)

        def a_rdma(d, src_slot):
            return pltpu.make_async_remote_copy(
                src_ref=amax_ref.at[me],
                dst_ref=amax_ref.at[src_slot],
                send_sem=as_sem.at[d],
                recv_sem=ar_sem.at[src_slot],
                device_id=(d,),
                device_id_type=pl.DeviceIdType.MESH,
            )

        for off in range(1, n_dev):
            d = lax.rem(me + off, n_dev)
            a_rdma(d, me).start()
        for off in range(1, n_dev):
            s = lax.rem(me + off, n_dev)
            a_rdma(me, s).wait_recv()
        for off in range(1, n_dev):
            d = lax.rem(me + off, n_dev)
            a_rdma(d, me).wait_send()

        g_amax = jnp.max(amax_ref[:, :, :])
        scale = g_amax / 448.0
        inv = 448.0 / g_amax
        q = (out_ref[:, :] * inv).astype(jnp.float8_e4m3fn)
        out_ref[:, :] = q.astype(jnp.float32) * scale

    return pl.pallas_call(
        body,
        out_shape=jax.ShapeDtypeStruct((m_per, n), jnp.float32),
        in_specs=[
            pl.BlockSpec(memory_space=pltpu.VMEM),
            pl.BlockSpec(memory_space=pl.ANY),
        ],
        out_specs=pl.BlockSpec(memory_space=pltpu.VMEM),
        scratch_shapes=[
            pltpu.VMEM((m_per, k_glob), jnp.float32),
            pltpu.VMEM((n_dev, 8, 128), jnp.float32),
            pltpu.VMEM((B, ks, nc), jnp.float32),
            pltpu.SemaphoreType.DMA((B, CH)),
            pltpu.SemaphoreType.DMA((n_dev,)),
            pltpu.SemaphoreType.DMA((n_dev,)),
            pltpu.SemaphoreType.DMA((n_dev,)),
            pltpu.SemaphoreType.DMA((n_dev,)),
        ],
        compiler_params=pltpu.CompilerParams(
            vmem_limit_bytes=100 * 1024 * 1024,
            collective_id=0,
        ),
    )(x, w_mat)
